# Initial kernel scaffold; baseline (speedup 1.0000x reference)
import jax
import jax.numpy as jnp
from jax.experimental import pallas as pl


def kernel(idx, targets, table):
    raise NotImplementedError("write your pallas kernel here")


# speedup vs baseline: 2.1791x; 2.1791x over previous
import functools

import jax
import jax.numpy as jnp
from jax import lax
from jax.experimental import pallas as pl
from jax.experimental.pallas import tpu as pltpu
from jax.experimental.pallas import tpu_sc as plsc

V = 1000
B, T = 1024, 200
TOK = B * T
NW = 32
PER_W = TOK // NW
CHL = 128
NCH = PER_W // CHL
L = 16
TB = 512


def _lse_body(table_ref, lse_ref):
    t = table_ref[...]
    m = jnp.max(t, axis=1)
    s = jnp.sum(jnp.exp(t - m[:, None]), axis=1)
    lse_ref[0, :] = m + jnp.log(s)


def _row_lse(table):
    out = pl.pallas_call(
        _lse_body,
        out_shape=jax.ShapeDtypeStruct((1, V), jnp.float32),
        in_specs=[pl.BlockSpec((V, V), lambda: (0, 0))],
        out_specs=pl.BlockSpec((1, V), lambda: (0, 0)),
    )(table)
    return out.reshape(V)


def _mm_body(idx_ref, tbl_ref, out_ref):
    ids = idx_ref[0]
    col = lax.broadcasted_iota(jnp.int32, (TB, V), 1)
    oh = (ids == col).astype(jnp.bfloat16)
    out_ref[...] = lax.dot_general(
        oh, tbl_ref[...], (((1,), (0,)), ((), ())),
        preferred_element_type=jnp.float32)


def _logits_matmul(idx_flat, table_bf):
    nblk = TOK // TB
    idx3 = idx_flat.reshape(nblk, TB, 1)
    return pl.pallas_call(
        _mm_body,
        grid=(nblk,),
        out_shape=jax.ShapeDtypeStruct((TOK, V), jnp.float32),
        in_specs=[
            pl.BlockSpec((1, TB, 1), lambda g: (g, 0, 0)),
            pl.BlockSpec((V, V), lambda g: (0, 0)),
        ],
        out_specs=pl.BlockSpec((TB, V), lambda g: (g, 0)),
    )(idx3, table_bf)


def _make_sc_loss():
    mesh = plsc.VectorSubcoreMesh(core_axis_name="c", subcore_axis_name="s")

    @functools.partial(
        pl.kernel,
        out_type=jax.ShapeDtypeStruct((NW, L), jnp.float32),
        mesh=mesh,
        scratch_types=[
            pltpu.VMEM((PER_W,), jnp.int32),
            pltpu.VMEM((PER_W,), jnp.int32),
            pltpu.VMEM((2, CHL), jnp.float32),
            pltpu.VMEM((2, CHL), jnp.float32),
            pltpu.VMEM((L,), jnp.float32),
            pltpu.SemaphoreType.DMA,
            pltpu.SemaphoreType.DMA,
        ],
    )
    def sc_loss(tflat_hbm, lse_hbm, idx_hbm, tgt_hbm, part_hbm,
                idx_v, fid_v, val_v, lseg_v, acc_v, sem0, sem1):
        wid = lax.axis_index("s") * 2 + lax.axis_index("c")
        base = wid * PER_W
        pltpu.sync_copy(idx_hbm.at[pl.ds(base, PER_W)], idx_v)
        pltpu.sync_copy(tgt_hbm.at[pl.ds(base, PER_W)], fid_v)

        def mkflat(i, _):
            o = i * L
            fid_v[pl.ds(o, L)] = idx_v[pl.ds(o, L)] * V + fid_v[pl.ds(o, L)]
            return 0

        lax.fori_loop(0, PER_W // L, mkflat, 0)

        sems = (sem0, sem1)

        def start(c, b):
            off = c * CHL
            pltpu.async_copy(
                tflat_hbm.at[fid_v.at[pl.ds(off, CHL)]], val_v.at[b], sems[b])
            pltpu.async_copy(
                lse_hbm.at[idx_v.at[pl.ds(off, CHL)]], lseg_v.at[b], sems[b])

        def wait(b):
            pltpu.make_async_copy(
                tflat_hbm.at[fid_v.at[pl.ds(0, CHL)]],
                val_v.at[b], sems[b]).wait()
            pltpu.make_async_copy(
                lse_hbm.at[idx_v.at[pl.ds(0, CHL)]],
                lseg_v.at[b], sems[b]).wait()

        start(0, 0)
        start(1, 1)

        def body(g, acc):
            for b in range(2):
                c = 2 * g + b
                wait(b)
                for sub in range(CHL // L):
                    o = sub * L
                    acc = acc + (lseg_v[b, pl.ds(o, L)] - val_v[b, pl.ds(o, L)])

                @pl.when(c + 2 < NCH)
                def _():
                    start(c + 2, b)
            return acc

        acc = lax.fori_loop(0, NCH // 2, body, jnp.zeros((L,), jnp.float32))
        acc_v[...] = acc
        pltpu.sync_copy(acc_v, part_hbm.at[wid])

    return sc_loss


_SC_LOSS = _make_sc_loss()


def kernel(idx, targets, table):
    idx_flat = idx.reshape(-1).astype(jnp.int32)
    tgt_flat = targets.reshape(-1).astype(jnp.int32)
    lse = _row_lse(table)
    partials = _SC_LOSS(table.reshape(-1), lse, idx_flat, tgt_flat)
    logits_flat = _logits_matmul(idx_flat, table.astype(jnp.bfloat16))
    logits = logits_flat.reshape(B, T, V)
    loss = jnp.sum(partials) / jnp.float32(TOK)
    return (logits, loss)
